# Initial kernel scaffold; baseline (speedup 1.0000x reference)
import jax
import jax.numpy as jnp
from jax.experimental import pallas as pl


def kernel(h, edge_index, edge_attr, We1, be1, We2, be2, Wn1, bn1, Wn2, bn2):
    raise NotImplementedError("write your pallas kernel here")


# speedup vs baseline: 3.4266x; 3.4266x over previous
import functools

import jax
import jax.numpy as jnp
from jax import lax
from jax.experimental import pallas as pl
from jax.experimental.pallas import tpu as pltpu
from jax.experimental.pallas import tpu_sc as plsc

_NC = 2
_NS = 16
_NW = _NC * _NS
_CH = 80


def _pre_body(h_ref, ws_ref, wt_ref, hs_ref, ht_ref):
    hb = h_ref[...]
    hs_ref[...] = jnp.dot(hb, ws_ref[...], preferred_element_type=jnp.float32)
    ht_ref[...] = jnp.dot(hb, wt_ref[...], preferred_element_type=jnp.float32)


def _edge_body(g_ref, ea_ref, wee_ref, be1_ref, we2_ref, be2_ref, mij_ref):
    x = g_ref[...] + jnp.dot(ea_ref[...], wee_ref[...],
                             preferred_element_type=jnp.float32) + be1_ref[...]
    x = x * lax.logistic(x)
    y = jnp.dot(x, we2_ref[...], preferred_element_type=jnp.float32) + be2_ref[...]
    mij_ref[...] = y * lax.logistic(y)


def _node_body(h_ref, p_ref, w1h_ref, w1a_ref, bn1_ref, wn2_ref, bn2_ref, o_ref):
    agg = p_ref[0] + p_ref[1]
    hb = h_ref[...]
    x = (jnp.dot(hb, w1h_ref[...], preferred_element_type=jnp.float32)
         + jnp.dot(agg, w1a_ref[...], preferred_element_type=jnp.float32)
         + bn1_ref[...])
    x = x * lax.logistic(x)
    o_ref[...] = hb + jnp.dot(x, wn2_ref[...],
                              preferred_element_type=jnp.float32) + bn2_ref[...]


def _make_sc_gather(N, E, H):
    per_w = E // _NW
    n_ch = per_w // _CH
    mesh = plsc.VectorSubcoreMesh(core_axis_name="c", subcore_axis_name="s")

    @functools.partial(
        pl.kernel,
        out_type=jax.ShapeDtypeStruct((E, H), jnp.float32),
        mesh=mesh,
        scratch_types=[
            pltpu.VMEM((n_ch, _CH), jnp.int32),
            pltpu.VMEM((n_ch, _CH), jnp.int32),
            pltpu.VMEM((_CH, H), jnp.float32),
            pltpu.SemaphoreType.DMA,
            pltpu.SemaphoreType.DMA,
        ],
    )
    def sc_gather(hs_hbm, ht_hbm, row3_hbm, col3_hbm, g_hbm,
                  idxr_v, idxc_v, bufa, sem1, sem2):
        w = lax.axis_index("s") * _NC + lax.axis_index("c")
        pltpu.sync_copy(row3_hbm.at[w], idxr_v)
        pltpu.sync_copy(col3_hbm.at[w], idxc_v)
        ebase = w * per_w

        def body(j, carry):
            cpa = pltpu.async_copy(hs_hbm.at[idxr_v.at[j]], bufa, sem1)
            cpa.wait()
            cpb = pltpu.async_copy(ht_hbm.at[idxc_v.at[j]], bufa, sem2, add=True)
            cpb.wait()
            pltpu.sync_copy(bufa, g_hbm.at[pl.ds(ebase + j * _CH, _CH)])
            return carry

        lax.fori_loop(0, n_ch, body, 0)

    return sc_gather


def _make_sc_scatter(N, E, H):
    per_w = E // _NW
    n_ch = per_w // _CH
    chz = 200
    n_z = N // chz
    z_per_tile = pl.cdiv(n_z, _NS)
    mesh = plsc.VectorSubcoreMesh(core_axis_name="c", subcore_axis_name="s")

    @functools.partial(
        pl.kernel,
        out_type=jax.ShapeDtypeStruct((_NC, N, H), jnp.float32),
        mesh=mesh,
        scratch_types=[
            pltpu.VMEM((n_ch, _CH), jnp.int32),
            pltpu.VMEM((_CH, H), jnp.float32),
            pltpu.VMEM_SHARED((N, H), jnp.float32),
        ],
    )
    def sc_scatter(mij_hbm, row3_hbm, zeros_hbm, out_hbm, idx_v, buf, acc_sh):
        c = lax.axis_index("c")
        s = lax.axis_index("s")

        def zbody(j, carry):
            ch = s + j * _NS

            @pl.when(ch < n_z)
            def _():
                pltpu.sync_copy(zeros_hbm.at[pl.ds(ch * chz, chz)],
                                acc_sh.at[pl.ds(ch * chz, chz)])
            return carry

        lax.fori_loop(0, z_per_tile, zbody, 0)
        plsc.subcore_barrier()
        wid = c * _NS + s
        pltpu.sync_copy(row3_hbm.at[wid], idx_v)
        ebase = wid * per_w

        def body(j, carry):
            pltpu.sync_copy(mij_hbm.at[pl.ds(ebase + j * _CH, _CH)], buf)
            pltpu.sync_copy(buf, acc_sh.at[idx_v.at[j]], add=True)
            return carry

        lax.fori_loop(0, n_ch, body, 0)
        plsc.subcore_barrier()

        def obody(j, carry):
            ch = s + j * _NS

            @pl.when(ch < n_z)
            def _():
                pltpu.sync_copy(acc_sh.at[pl.ds(ch * chz, chz)],
                                out_hbm.at[c, pl.ds(ch * chz, chz)])
            return carry

        lax.fori_loop(0, z_per_tile, obody, 0)

    return sc_scatter


def kernel(h, edge_index, edge_attr, We1, be1, We2, be2, Wn1, bn1, Wn2, bn2):
    N, D = h.shape
    E = edge_index.shape[1]
    H = We2.shape[1]

    row = edge_index[0].astype(jnp.int32)
    col = edge_index[1].astype(jnp.int32)
    n_ch = E // _NW // _CH
    row3 = row.reshape(_NW, n_ch, _CH)
    col3 = col.reshape(_NW, n_ch, _CH)

    Ws = We1[:D]
    Wt = We1[D:2 * D]
    Wee = We1[2 * D:]
    be1r = be1.reshape(1, H)
    be2r = be2.reshape(1, H)
    bn1r = bn1.reshape(1, H)
    bn2r = bn2.reshape(1, D)
    W1h = Wn1[:D]
    W1a = Wn1[D:]

    BN = 2000
    hs, ht = pl.pallas_call(
        _pre_body,
        grid=(N // BN,),
        in_specs=[
            pl.BlockSpec((BN, D), lambda i: (i, 0)),
            pl.BlockSpec((D, H), lambda i: (0, 0)),
            pl.BlockSpec((D, H), lambda i: (0, 0)),
        ],
        out_specs=[
            pl.BlockSpec((BN, H), lambda i: (i, 0)),
            pl.BlockSpec((BN, H), lambda i: (i, 0)),
        ],
        out_shape=[
            jax.ShapeDtypeStruct((N, H), jnp.float32),
            jax.ShapeDtypeStruct((N, H), jnp.float32),
        ],
    )(h, Ws, Wt)

    g = _make_sc_gather(N, E, H)(hs, ht, row3, col3)

    BE = 2000
    DE = edge_attr.shape[1]
    mij = pl.pallas_call(
        _edge_body,
        grid=(E // BE,),
        in_specs=[
            pl.BlockSpec((BE, H), lambda i: (i, 0)),
            pl.BlockSpec((BE, DE), lambda i: (i, 0)),
            pl.BlockSpec((DE, H), lambda i: (0, 0)),
            pl.BlockSpec((1, H), lambda i: (0, 0)),
            pl.BlockSpec((H, H), lambda i: (0, 0)),
            pl.BlockSpec((1, H), lambda i: (0, 0)),
        ],
        out_specs=pl.BlockSpec((BE, H), lambda i: (i, 0)),
        out_shape=jax.ShapeDtypeStruct((E, H), jnp.float32),
    )(g, edge_attr, Wee, be1r, We2, be2r)

    zeros_nh = jnp.zeros((N, H), jnp.float32)
    partials = _make_sc_scatter(N, E, H)(mij, row3, zeros_nh)

    h_out = pl.pallas_call(
        _node_body,
        grid=(N // BN,),
        in_specs=[
            pl.BlockSpec((BN, D), lambda i: (i, 0)),
            pl.BlockSpec((_NC, BN, H), lambda i: (0, i, 0)),
            pl.BlockSpec((D, H), lambda i: (0, 0)),
            pl.BlockSpec((H, H), lambda i: (0, 0)),
            pl.BlockSpec((1, H), lambda i: (0, 0)),
            pl.BlockSpec((H, D), lambda i: (0, 0)),
            pl.BlockSpec((1, D), lambda i: (0, 0)),
        ],
        out_specs=pl.BlockSpec((BN, D), lambda i: (i, 0)),
        out_shape=jax.ShapeDtypeStruct((N, D), jnp.float32),
    )(h, partials, W1h, W1a, bn1r, Wn2, bn2r)

    return (h_out, mij)


# speedup vs baseline: 4.4256x; 1.2915x over previous
import functools

import jax
import jax.numpy as jnp
from jax import lax
from jax.experimental import pallas as pl
from jax.experimental.pallas import tpu as pltpu
from jax.experimental.pallas import tpu_sc as plsc

_NC = 2
_NS = 16
_NW = _NC * _NS
_CH = 80
_CHS = 40


def _pre_body(h_ref, ws_ref, wt_ref, hs_ref, ht_ref):
    hb = h_ref[...]
    hs_ref[...] = jnp.dot(hb, ws_ref[...], preferred_element_type=jnp.float32)
    ht_ref[...] = jnp.dot(hb, wt_ref[...], preferred_element_type=jnp.float32)


def _edge_body(g_ref, ea_ref, wee_ref, be1_ref, we2_ref, be2_ref, mij_ref):
    x = g_ref[...] + jnp.dot(ea_ref[...], wee_ref[...],
                             preferred_element_type=jnp.float32) + be1_ref[...]
    x = x * lax.logistic(x)
    y = jnp.dot(x, we2_ref[...], preferred_element_type=jnp.float32) + be2_ref[...]
    mij_ref[...] = y * lax.logistic(y)


def _node_body(h_ref, p_ref, w1h_ref, w1a_ref, bn1_ref, wn2_ref, bn2_ref, o_ref):
    agg = p_ref[0] + p_ref[1]
    hb = h_ref[...]
    x = (jnp.dot(hb, w1h_ref[...], preferred_element_type=jnp.float32)
         + jnp.dot(agg, w1a_ref[...], preferred_element_type=jnp.float32)
         + bn1_ref[...])
    x = x * lax.logistic(x)
    o_ref[...] = hb + jnp.dot(x, wn2_ref[...],
                              preferred_element_type=jnp.float32) + bn2_ref[...]


_K = 5


def _make_sc_gather(N, E, H):
    per_w = E // _NW
    n_ch = per_w // _CH
    n_rounds = n_ch // _K
    mesh = plsc.VectorSubcoreMesh(core_axis_name="c", subcore_axis_name="s")

    @functools.partial(
        pl.kernel,
        out_type=jax.ShapeDtypeStruct((E, H), jnp.float32),
        mesh=mesh,
        scratch_types=[
            pltpu.VMEM((_K, _CH), jnp.int32),
            pltpu.VMEM((_K, _CH), jnp.int32),
            pltpu.VMEM((_K, _CH, H), jnp.float32),
        ] + [pltpu.SemaphoreType.DMA] * (3 * _K),
    )
    def sc_gather(hs_hbm, ht_hbm, row4_hbm, col4_hbm, g_hbm,
                  idxr_v, idxc_v, bufs, *sems):
        semg = sems[0:_K]
        sema = sems[_K:2 * _K]
        semw = sems[2 * _K:3 * _K]
        w = lax.axis_index("s") * _NC + lax.axis_index("c")
        ebase = w * per_w

        def round_body(t, carry):
            j0 = t * _K
            pltpu.sync_copy(row4_hbm.at[w, t], idxr_v)
            pltpu.sync_copy(col4_hbm.at[w, t], idxc_v)
            cps_a = []
            for b in range(_K):
                @pl.when(t > 0)
                def _(b=b):
                    pltpu.make_async_copy(
                        bufs.at[b], g_hbm.at[pl.ds(ebase, _CH)], semw[b]
                    ).wait()
                cps_a.append(pltpu.async_copy(
                    hs_hbm.at[idxr_v.at[b]], bufs.at[b], semg[b]))
            cps_b = []
            for b in range(_K):
                cps_a[b].wait()
                cps_b.append(pltpu.async_copy(
                    ht_hbm.at[idxc_v.at[b]], bufs.at[b], sema[b], add=True))
            for b in range(_K):
                cps_b[b].wait()
                pltpu.async_copy(
                    bufs.at[b],
                    g_hbm.at[pl.ds(ebase + (j0 + b) * _CH, _CH)], semw[b])
            return carry

        lax.fori_loop(0, n_rounds, round_body, 0)
        for b in range(_K):
            pltpu.make_async_copy(
                bufs.at[b], g_hbm.at[pl.ds(ebase, _CH)], semw[b]).wait()

    return sc_gather


def _make_sc_scatter(N, E, H):
    per_w = E // _NW
    n_ch = per_w // _CHS
    chz = 200
    n_z = N // chz
    z_per_tile = pl.cdiv(n_z, _NS)
    mesh = plsc.VectorSubcoreMesh(core_axis_name="c", subcore_axis_name="s")

    @functools.partial(
        pl.kernel,
        out_type=jax.ShapeDtypeStruct((_NC, N, H), jnp.float32),
        mesh=mesh,
        scratch_types=[
            pltpu.VMEM((_K, _CHS), jnp.int32),
            pltpu.VMEM((_K, _CHS, H), jnp.float32),
            pltpu.VMEM_SHARED((N, H), jnp.float32),
        ] + [pltpu.SemaphoreType.DMA] * (2 * _K),
    )
    def sc_scatter(mij_hbm, row4_hbm, zeros_hbm, out_hbm, idx_v, bufs, acc_sh,
                   *sems):
        semr = sems[0:_K]
        semsc = sems[_K:2 * _K]
        c = lax.axis_index("c")
        s = lax.axis_index("s")

        def zbody(j, carry):
            ch = s + j * _NS

            @pl.when(ch < n_z)
            def _():
                pltpu.sync_copy(zeros_hbm.at[pl.ds(ch * chz, chz)],
                                acc_sh.at[pl.ds(ch * chz, chz)])
            return carry

        lax.fori_loop(0, z_per_tile, zbody, 0)
        plsc.subcore_barrier()
        wid = c * _NS + s
        ebase = wid * per_w

        n_rounds = n_ch // _K

        def round_body(t, carry):
            j0 = t * _K
            pltpu.sync_copy(row4_hbm.at[wid, t], idx_v)
            cps = []
            for b in range(_K):
                @pl.when(t > 0)
                def _(b=b):
                    pltpu.make_async_copy(
                        bufs.at[b], acc_sh.at[idx_v.at[0]], semsc[b]).wait()
                cps.append(pltpu.async_copy(
                    mij_hbm.at[pl.ds(ebase + (j0 + b) * _CHS, _CHS)],
                    bufs.at[b], semr[b]))
            for b in range(_K):
                cps[b].wait()
                pltpu.async_copy(
                    bufs.at[b], acc_sh.at[idx_v.at[b]], semsc[b], add=True)
            return carry

        lax.fori_loop(0, n_rounds, round_body, 0)
        for b in range(_K):
            pltpu.make_async_copy(
                bufs.at[b], acc_sh.at[idx_v.at[0]], semsc[b]).wait()
        plsc.subcore_barrier()

        def obody(j, carry):
            ch = s + j * _NS

            @pl.when(ch < n_z)
            def _():
                pltpu.sync_copy(acc_sh.at[pl.ds(ch * chz, chz)],
                                out_hbm.at[c, pl.ds(ch * chz, chz)])
            return carry

        lax.fori_loop(0, z_per_tile, obody, 0)

    return sc_scatter


def kernel(h, edge_index, edge_attr, We1, be1, We2, be2, Wn1, bn1, Wn2, bn2):
    N, D = h.shape
    E = edge_index.shape[1]
    H = We2.shape[1]

    row = edge_index[0].astype(jnp.int32)
    col = edge_index[1].astype(jnp.int32)
    n_rounds = E // _NW // _CH // _K
    row4 = row.reshape(_NW, n_rounds, _K, _CH)
    col4 = col.reshape(_NW, n_rounds, _K, _CH)
    row4s = row.reshape(_NW, E // _NW // _CHS // _K, _K, _CHS)

    Ws = We1[:D]
    Wt = We1[D:2 * D]
    Wee = We1[2 * D:]
    be1r = be1.reshape(1, H)
    be2r = be2.reshape(1, H)
    bn1r = bn1.reshape(1, H)
    bn2r = bn2.reshape(1, D)
    W1h = Wn1[:D]
    W1a = Wn1[D:]

    BN = 2000
    hs, ht = pl.pallas_call(
        _pre_body,
        grid=(N // BN,),
        in_specs=[
            pl.BlockSpec((BN, D), lambda i: (i, 0)),
            pl.BlockSpec((D, H), lambda i: (0, 0)),
            pl.BlockSpec((D, H), lambda i: (0, 0)),
        ],
        out_specs=[
            pl.BlockSpec((BN, H), lambda i: (i, 0)),
            pl.BlockSpec((BN, H), lambda i: (i, 0)),
        ],
        out_shape=[
            jax.ShapeDtypeStruct((N, H), jnp.float32),
            jax.ShapeDtypeStruct((N, H), jnp.float32),
        ],
    )(h, Ws, Wt)

    g = _make_sc_gather(N, E, H)(hs, ht, row4, col4)

    BE = 2000
    DE = edge_attr.shape[1]
    mij = pl.pallas_call(
        _edge_body,
        grid=(E // BE,),
        in_specs=[
            pl.BlockSpec((BE, H), lambda i: (i, 0)),
            pl.BlockSpec((BE, DE), lambda i: (i, 0)),
            pl.BlockSpec((DE, H), lambda i: (0, 0)),
            pl.BlockSpec((1, H), lambda i: (0, 0)),
            pl.BlockSpec((H, H), lambda i: (0, 0)),
            pl.BlockSpec((1, H), lambda i: (0, 0)),
        ],
        out_specs=pl.BlockSpec((BE, H), lambda i: (i, 0)),
        out_shape=jax.ShapeDtypeStruct((E, H), jnp.float32),
    )(g, edge_attr, Wee, be1r, We2, be2r)

    zeros_nh = jnp.zeros((N, H), jnp.float32)
    partials = _make_sc_scatter(N, E, H)(mij, row4s, zeros_nh)

    h_out = pl.pallas_call(
        _node_body,
        grid=(N // BN,),
        in_specs=[
            pl.BlockSpec((BN, D), lambda i: (i, 0)),
            pl.BlockSpec((_NC, BN, H), lambda i: (0, i, 0)),
            pl.BlockSpec((D, H), lambda i: (0, 0)),
            pl.BlockSpec((H, H), lambda i: (0, 0)),
            pl.BlockSpec((1, H), lambda i: (0, 0)),
            pl.BlockSpec((H, D), lambda i: (0, 0)),
            pl.BlockSpec((1, D), lambda i: (0, 0)),
        ],
        out_specs=pl.BlockSpec((BN, D), lambda i: (i, 0)),
        out_shape=jax.ShapeDtypeStruct((N, D), jnp.float32),
    )(h, partials, W1h, W1a, bn1r, Wn2, bn2r)

    return (h_out, mij)


# speedup vs baseline: 4.6897x; 1.0597x over previous
import functools

import jax
import jax.numpy as jnp
from jax import lax
from jax.experimental import pallas as pl
from jax.experimental.pallas import tpu as pltpu
from jax.experimental.pallas import tpu_sc as plsc

_NC = 2
_NS = 16
_NW = _NC * _NS
_CH = 80
_CHS = 40
_P = 5


def _pre_body(h_ref, ws_ref, wt_ref, hs_ref, ht_ref):
    hb = h_ref[...]
    hs_ref[...] = jnp.dot(hb, ws_ref[...], preferred_element_type=jnp.float32)
    ht_ref[...] = jnp.dot(hb, wt_ref[...], preferred_element_type=jnp.float32)


def _edge_body(g_ref, ea_ref, wee_ref, be1_ref, we2_ref, be2_ref, mij_ref):
    x = g_ref[...] + jnp.dot(ea_ref[...], wee_ref[...],
                             preferred_element_type=jnp.float32) + be1_ref[...]
    x = x * lax.logistic(x)
    y = jnp.dot(x, we2_ref[...], preferred_element_type=jnp.float32) + be2_ref[...]
    mij_ref[...] = y * lax.logistic(y)


def _edge_body_acc(prev_ref, g_ref, ea_ref, wee_ref, be1_ref, we2_ref,
                   be2_ref, mij_ref):
    del prev_ref
    _edge_body(g_ref, ea_ref, wee_ref, be1_ref, we2_ref, be2_ref, mij_ref)


def _node_body(h_ref, p_ref, w1h_ref, w1a_ref, bn1_ref, wn2_ref, bn2_ref, o_ref):
    agg = p_ref[0] + p_ref[1]
    hb = h_ref[...]
    x = (jnp.dot(hb, w1h_ref[...], preferred_element_type=jnp.float32)
         + jnp.dot(agg, w1a_ref[...], preferred_element_type=jnp.float32)
         + bn1_ref[...])
    x = x * lax.logistic(x)
    o_ref[...] = hb + jnp.dot(x, wn2_ref[...],
                              preferred_element_type=jnp.float32) + bn2_ref[...]


_K = 5


def _make_sc_gather(N, E, H):
    per_w = E // _NW
    n_ch = per_w // _CH
    n_rounds = n_ch // _K
    mesh = plsc.VectorSubcoreMesh(core_axis_name="c", subcore_axis_name="s")

    @functools.partial(
        pl.kernel,
        out_type=jax.ShapeDtypeStruct((E, H), jnp.float32),
        mesh=mesh,
        scratch_types=[
            pltpu.VMEM((_K, _CH), jnp.int32),
            pltpu.VMEM((_K, _CH), jnp.int32),
            pltpu.VMEM((_K, _CH, H), jnp.float32),
        ] + [pltpu.SemaphoreType.DMA] * (3 * _K),
    )
    def sc_gather(hs_hbm, ht_hbm, row4_hbm, col4_hbm, g_hbm,
                  idxr_v, idxc_v, bufs, *sems):
        semg = sems[0:_K]
        sema = sems[_K:2 * _K]
        semw = sems[2 * _K:3 * _K]
        w = lax.axis_index("s") * _NC + lax.axis_index("c")
        ebase = w * per_w

        def round_body(t, carry):
            j0 = t * _K
            pltpu.sync_copy(row4_hbm.at[w, t], idxr_v)
            pltpu.sync_copy(col4_hbm.at[w, t], idxc_v)
            cps_a = []
            for b in range(_K):
                @pl.when(t > 0)
                def _(b=b):
                    pltpu.make_async_copy(
                        bufs.at[b], g_hbm.at[pl.ds(ebase, _CH)], semw[b]
                    ).wait()
                cps_a.append(pltpu.async_copy(
                    hs_hbm.at[idxr_v.at[b]], bufs.at[b], semg[b]))
            cps_b = []
            for b in range(_K):
                cps_a[b].wait()
                cps_b.append(pltpu.async_copy(
                    ht_hbm.at[idxc_v.at[b]], bufs.at[b], sema[b], add=True))
            for b in range(_K):
                cps_b[b].wait()
                pltpu.async_copy(
                    bufs.at[b],
                    g_hbm.at[pl.ds(ebase + (j0 + b) * _CH, _CH)], semw[b])
            return carry

        lax.fori_loop(0, n_rounds, round_body, 0)
        for b in range(_K):
            pltpu.make_async_copy(
                bufs.at[b], g_hbm.at[pl.ds(ebase, _CH)], semw[b]).wait()

    return sc_gather


def _make_sc_scatter(N, E, H):
    per_w = E // _NW
    n_ch = per_w // _CHS
    chz = 200
    n_z = N // chz
    z_per_tile = pl.cdiv(n_z, _NS)
    mesh = plsc.VectorSubcoreMesh(core_axis_name="c", subcore_axis_name="s")

    @functools.partial(
        pl.kernel,
        out_type=jax.ShapeDtypeStruct((_NC, N, H), jnp.float32),
        mesh=mesh,
        scratch_types=[
            pltpu.VMEM((_K, _CHS), jnp.int32),
            pltpu.VMEM((_K, _CHS, H), jnp.float32),
            pltpu.VMEM_SHARED((N, H), jnp.float32),
        ] + [pltpu.SemaphoreType.DMA] * (2 * _K),
    )
    def sc_scatter(mij_hbm, row4_hbm, zeros_hbm, out_hbm, idx_v, bufs, acc_sh,
                   *sems):
        semr = sems[0:_K]
        semsc = sems[_K:2 * _K]
        c = lax.axis_index("c")
        s = lax.axis_index("s")

        def zbody(j, carry):
            ch = s + j * _NS

            @pl.when(ch < n_z)
            def _():
                pltpu.sync_copy(zeros_hbm.at[pl.ds(ch * chz, chz)],
                                acc_sh.at[pl.ds(ch * chz, chz)])
            return carry

        lax.fori_loop(0, z_per_tile, zbody, 0)
        plsc.subcore_barrier()
        wid = c * _NS + s
        ebase = wid * per_w

        n_rounds = n_ch // _K

        def round_body(t, carry):
            j0 = t * _K
            pltpu.sync_copy(row4_hbm.at[wid, t], idx_v)
            cps = []
            for b in range(_K):
                @pl.when(t > 0)
                def _(b=b):
                    pltpu.make_async_copy(
                        bufs.at[b], acc_sh.at[idx_v.at[0]], semsc[b]).wait()
                cps.append(pltpu.async_copy(
                    mij_hbm.at[pl.ds(ebase + (j0 + b) * _CHS, _CHS)],
                    bufs.at[b], semr[b]))
            for b in range(_K):
                cps[b].wait()
                pltpu.async_copy(
                    bufs.at[b], acc_sh.at[idx_v.at[b]], semsc[b], add=True)
            return carry

        lax.fori_loop(0, n_rounds, round_body, 0)
        for b in range(_K):
            pltpu.make_async_copy(
                bufs.at[b], acc_sh.at[idx_v.at[0]], semsc[b]).wait()
        plsc.subcore_barrier()

        def obody(j, carry):
            ch = s + j * _NS

            @pl.when(ch < n_z)
            def _():
                pltpu.sync_copy(acc_sh.at[pl.ds(ch * chz, chz)],
                                out_hbm.at[c, pl.ds(ch * chz, chz)])
            return carry

        lax.fori_loop(0, z_per_tile, obody, 0)

    return sc_scatter


def kernel(h, edge_index, edge_attr, We1, be1, We2, be2, Wn1, bn1, Wn2, bn2):
    N, D = h.shape
    E = edge_index.shape[1]
    H = We2.shape[1]

    row = edge_index[0].astype(jnp.int32)
    col = edge_index[1].astype(jnp.int32)
    Es = E // _P
    n_rounds = Es // _NW // _CH // _K
    row5 = row.reshape(_P, _NW, n_rounds, _K, _CH)
    col5 = col.reshape(_P, _NW, n_rounds, _K, _CH)
    row4s = row.reshape(_NW, E // _NW // _CHS // _K, _K, _CHS)

    Ws = We1[:D]
    Wt = We1[D:2 * D]
    Wee = We1[2 * D:]
    be1r = be1.reshape(1, H)
    be2r = be2.reshape(1, H)
    bn1r = bn1.reshape(1, H)
    bn2r = bn2.reshape(1, D)
    W1h = Wn1[:D]
    W1a = Wn1[D:]

    BN = 2000
    hs, ht = pl.pallas_call(
        _pre_body,
        grid=(N // BN,),
        in_specs=[
            pl.BlockSpec((BN, D), lambda i: (i, 0)),
            pl.BlockSpec((D, H), lambda i: (0, 0)),
            pl.BlockSpec((D, H), lambda i: (0, 0)),
        ],
        out_specs=[
            pl.BlockSpec((BN, H), lambda i: (i, 0)),
            pl.BlockSpec((BN, H), lambda i: (i, 0)),
        ],
        out_shape=[
            jax.ShapeDtypeStruct((N, H), jnp.float32),
            jax.ShapeDtypeStruct((N, H), jnp.float32),
        ],
    )(h, Ws, Wt)

    BE = 2000
    DE = edge_attr.shape[1]
    nbe = Es // BE
    gather_fn = _make_sc_gather(N, Es, H)
    g_slices = [gather_fn(hs, ht, row5[p], col5[p]) for p in range(_P)]

    mij = None
    for p in range(_P):
        in_specs = [
            pl.BlockSpec((BE, H), lambda i: (i, 0)),
            pl.BlockSpec((BE, DE), lambda i, p=p: (p * nbe + i, 0)),
            pl.BlockSpec((DE, H), lambda i: (0, 0)),
            pl.BlockSpec((1, H), lambda i: (0, 0)),
            pl.BlockSpec((H, H), lambda i: (0, 0)),
            pl.BlockSpec((1, H), lambda i: (0, 0)),
        ]
        out_spec = pl.BlockSpec((BE, H), lambda i, p=p: (p * nbe + i, 0))
        if p == 0:
            mij = pl.pallas_call(
                _edge_body,
                grid=(nbe,),
                in_specs=in_specs,
                out_specs=out_spec,
                out_shape=jax.ShapeDtypeStruct((E, H), jnp.float32),
            )(g_slices[0], edge_attr, Wee, be1r, We2, be2r)
        else:
            mij = pl.pallas_call(
                _edge_body_acc,
                grid=(nbe,),
                in_specs=[pl.BlockSpec(memory_space=pl.ANY)]
                + in_specs,
                out_specs=out_spec,
                out_shape=jax.ShapeDtypeStruct((E, H), jnp.float32),
                input_output_aliases={0: 0},
            )(mij, g_slices[p], edge_attr, Wee, be1r, We2, be2r)

    zeros_nh = jnp.zeros((N, H), jnp.float32)
    partials = _make_sc_scatter(N, E, H)(mij, row4s, zeros_nh)

    h_out = pl.pallas_call(
        _node_body,
        grid=(N // BN,),
        in_specs=[
            pl.BlockSpec((BN, D), lambda i: (i, 0)),
            pl.BlockSpec((_NC, BN, H), lambda i: (0, i, 0)),
            pl.BlockSpec((D, H), lambda i: (0, 0)),
            pl.BlockSpec((H, H), lambda i: (0, 0)),
            pl.BlockSpec((1, H), lambda i: (0, 0)),
            pl.BlockSpec((H, D), lambda i: (0, 0)),
            pl.BlockSpec((1, D), lambda i: (0, 0)),
        ],
        out_specs=pl.BlockSpec((BN, D), lambda i: (i, 0)),
        out_shape=jax.ShapeDtypeStruct((N, D), jnp.float32),
    )(h, partials, W1h, W1a, bn1r, Wn2, bn2r)

    return (h_out, mij)
